# baseline (speedup 1.0000x reference)
import functools

import jax
import jax.numpy as jnp
from jax import lax
from jax.experimental import pallas as pl
from jax.experimental.pallas import tpu as pltpu
from jax.experimental.pallas import tpu_sc as plsc

N = 10000
E = 160000
D = 256
DH = 128
L = 3
G = 64
BN_EPS = 1e-5

NSUB = 16
BLK = 128
IGRP = 8
NGRP = 11
BLOCKS_PER_TILE = NGRP * IGRP
EPT = BLOCKS_PER_TILE * BLK
EPAD = NSUB * EPT
NPAD = 10112
ZROWS = NPAD // NSUB

R = 2000


@functools.cache
def _make_agg_sc():
  mesh = plsc.VectorSubcoreMesh(core_axis_name="c", subcore_axis_name="s")

  @functools.partial(
      pl.kernel,
      mesh=mesh,
      out_type=[
          jax.ShapeDtypeStruct((NPAD, DH), jnp.float32),
          jax.ShapeDtypeStruct((NPAD, DH), jnp.float32),
      ],
      scratch_types=[
          pltpu.VMEM((BLOCKS_PER_TILE, BLK), jnp.int32),
          pltpu.VMEM((BLOCKS_PER_TILE, BLK), jnp.int32),
          pltpu.VMEM((BLK, DH), jnp.float32),
          pltpu.VMEM_SHARED((NPAD, DH), jnp.float32),
          pltpu.SemaphoreType.DMA,
      ],
  )
  def _agg_sc(h0_hbm, h1_hbm, src3_hbm, dst3_hbm, zeros_hbm, out0_hbm, out1_hbm,
              src_v, dst_v, rows0, acc_sh, sem_g):
    c = lax.axis_index("c")
    s = lax.axis_index("s")

    pltpu.sync_copy(zeros_hbm.at[pl.ds(s * ZROWS, ZROWS)],
                    acc_sh.at[pl.ds(s * ZROWS, ZROWS)])
    plsc.subcore_barrier()

    def body(h_hbm):
        pltpu.sync_copy(src3_hbm.at[s], src_v)
        pltpu.sync_copy(dst3_hbm.at[s], dst_v)

        def blk(j, carry):
            pltpu.async_copy(h_hbm.at[src_v.at[j]], rows0, sem_g).wait()
            pltpu.sync_copy(rows0, acc_sh.at[dst_v.at[j]], add=True)
            return carry

        lax.fori_loop(0, BLOCKS_PER_TILE, blk, 0)

    @pl.when(c == 0)
    def _():
        body(h0_hbm)

    @pl.when(c == 1)
    def _():
        body(h1_hbm)

    plsc.subcore_barrier()

    @pl.when(c == 0)
    def _():
        pltpu.sync_copy(acc_sh.at[pl.ds(s * ZROWS, ZROWS)],
                        out0_hbm.at[pl.ds(s * ZROWS, ZROWS)])

    @pl.when(c == 1)
    def _():
        pltpu.sync_copy(acc_sh.at[pl.ds(s * ZROWS, ZROWS)],
                        out1_hbm.at[pl.ds(s * ZROWS, ZROWS)])

  return _agg_sc


def _agg_call(h0, h1, src2, dst2, zeros):
    return _make_agg_sc()(h0, h1, src2, dst2, zeros)


def _mlp_body(h0, h1, a0, a1, w1, b1, w2, b2, out, ssum, ssq):
    x = jnp.concatenate([h0[...] + a0[...], h1[...] + a1[...]], axis=1)
    t = jnp.dot(x, w1[...], preferred_element_type=jnp.float32) + b1[...]
    t = jnp.maximum(t, 0.0)
    m = jnp.dot(t, w2[...], preferred_element_type=jnp.float32) + b2[...]
    m = jnp.maximum(m, 0.0)
    out[...] = m

    @pl.when(pl.program_id(0) == 0)
    def _():
        ssum[...] = jnp.zeros_like(ssum)
        ssq[...] = jnp.zeros_like(ssq)

    ssum[...] += jnp.sum(m, axis=0, keepdims=True)
    ssq[...] += jnp.sum(m * m, axis=0, keepdims=True)


def _mlp_call(h0, h1, a0, a1, w1, b1, w2, b2):
    row = pl.BlockSpec((R, DH), lambda i: (i, 0))
    full = pl.BlockSpec((D, D), lambda i: (0, 0))
    vec = pl.BlockSpec((1, D), lambda i: (0, 0))
    return pl.pallas_call(
        _mlp_body,
        grid=(N // R,),
        in_specs=[row, row, row, row, full, vec, full, vec],
        out_specs=[pl.BlockSpec((R, D), lambda i: (i, 0)), vec, vec],
        out_shape=[
            jax.ShapeDtypeStruct((N, D), jnp.float32),
            jax.ShapeDtypeStruct((1, D), jnp.float32),
            jax.ShapeDtypeStruct((1, D), jnp.float32),
        ],
    )(h0, h1, a0, a1, w1, b1, w2, b2)


def _bn_body(mraw, ssum, ssq, gamma, beta, batch, m0, m1, pool):
    mean = ssum[...] * (1.0 / N)
    var = ssq[...] * (1.0 / N) - mean * mean
    scale = gamma[...] * lax.rsqrt(var + BN_EPS)
    shift = beta[...] - mean * scale
    m = mraw[...] * scale + shift
    m0[...] = m[:, :DH]
    m1[...] = m[:, DH:]
    oh = (batch[...] == lax.broadcasted_iota(jnp.int32, (R, G), 1))
    oh = oh.astype(jnp.float32)

    @pl.when(pl.program_id(0) == 0)
    def _():
        pool[...] = jnp.zeros_like(pool)

    pool[...] += lax.dot_general(oh, m, (((0,), (0,)), ((), ())),
                                 preferred_element_type=jnp.float32)


def _bn_call(mraw, ssum, ssq, gamma, beta, batch2):
    vec = pl.BlockSpec((1, D), lambda i: (0, 0))
    return pl.pallas_call(
        _bn_body,
        grid=(N // R,),
        in_specs=[
            pl.BlockSpec((R, D), lambda i: (i, 0)),
            vec, vec, vec, vec,
            pl.BlockSpec((R, 1), lambda i: (i, 0)),
        ],
        out_specs=[
            pl.BlockSpec((R, DH), lambda i: (i, 0)),
            pl.BlockSpec((R, DH), lambda i: (i, 0)),
            pl.BlockSpec((G, D), lambda i: (0, 0)),
        ],
        out_shape=[
            jax.ShapeDtypeStruct((N, DH), jnp.float32),
            jax.ShapeDtypeStruct((N, DH), jnp.float32),
            jax.ShapeDtypeStruct((G, D), jnp.float32),
        ],
    )(mraw, ssum, ssq, gamma, beta, batch2)


def kernel(x, edge_index, batch,
           W1_0, b1_0, W2_0, b2_0, gamma_0, beta_0,
           W1_1, b1_1, W2_1, b2_1, gamma_1, beta_1,
           W1_2, b1_2, W2_2, b2_2, gamma_2, beta_2):
    src = edge_index[0].astype(jnp.int32)
    dst = edge_index[1].astype(jnp.int32)
    pad = EPAD - E
    src2 = jnp.concatenate([src, jnp.zeros((pad,), jnp.int32)]).reshape(
        NSUB, BLOCKS_PER_TILE, BLK)
    dst2 = jnp.concatenate([dst, jnp.full((pad,), N, jnp.int32)]).reshape(
        NSUB, BLOCKS_PER_TILE, BLK)
    zeros = jnp.zeros((NPAD, DH), jnp.float32)
    batch2 = batch.reshape(N, 1).astype(jnp.int32)

    params = [
        (W1_0, b1_0.reshape(1, D), W2_0, b2_0.reshape(1, D),
         gamma_0.reshape(1, D), beta_0.reshape(1, D)),
        (W1_1, b1_1.reshape(1, D), W2_1, b2_1.reshape(1, D),
         gamma_1.reshape(1, D), beta_1.reshape(1, D)),
        (W1_2, b1_2.reshape(1, D), W2_2, b2_2.reshape(1, D),
         gamma_2.reshape(1, D), beta_2.reshape(1, D)),
    ]

    h0 = x[:, :DH]
    h1 = x[:, DH:]
    halves = []
    pools = []
    for i in range(L):
        w1, b1, w2, b2, gmm, bta = params[i]
        a0, a1 = _agg_call(h0, h1, src2, dst2, zeros)
        mraw, ssum, ssq = _mlp_call(h0, h1, a0, a1, w1, b1, w2, b2)
        m0, m1, pool = _bn_call(mraw, ssum, ssq, gmm, bta, batch2)
        h0, h1 = m0, m1
        halves.extend([m0, m1])
        pools.append(pool)

    return jnp.concatenate(pools, axis=1), jnp.concatenate(halves, axis=1)


# speedup vs baseline: 3.0038x; 3.0038x over previous
import functools

import jax
import jax.numpy as jnp
from jax import lax
from jax.experimental import pallas as pl
from jax.experimental.pallas import tpu as pltpu
from jax.experimental.pallas import tpu_sc as plsc

N = 10000
E = 160000
D = 256
DH = 128
L = 3
G = 64
BN_EPS = 1e-5

NSUB = 16
BLK = 128
BLOCKS_PER_TILE = 80
EPT = BLOCKS_PER_TILE * BLK
EPAD = NSUB * EPT
NPAD = 10112
ZROWS = NPAD // NSUB

R = 2000


@functools.cache
def _make_agg_sc():
  mesh = plsc.VectorSubcoreMesh(core_axis_name="c", subcore_axis_name="s")

  @functools.partial(
      pl.kernel,
      mesh=mesh,
      out_type=[
          jax.ShapeDtypeStruct((NPAD, DH), jnp.float32),
          jax.ShapeDtypeStruct((NPAD, DH), jnp.float32),
      ],
      scratch_types=[
          pltpu.VMEM((BLOCKS_PER_TILE, BLK), jnp.int32),
          pltpu.VMEM((BLOCKS_PER_TILE, BLK), jnp.int32),
          pltpu.VMEM((BLK, DH), jnp.float32),
          pltpu.VMEM_SHARED((NPAD, DH), jnp.float32),
          pltpu.SemaphoreType.DMA,
      ],
  )
  def _agg_sc(h0_hbm, h1_hbm, src3_hbm, dst3_hbm, zeros_hbm, out0_hbm, out1_hbm,
              src_v, dst_v, rows0, acc_sh, sem_g):
    c = lax.axis_index("c")
    s = lax.axis_index("s")

    pltpu.sync_copy(zeros_hbm.at[pl.ds(s * ZROWS, ZROWS)],
                    acc_sh.at[pl.ds(s * ZROWS, ZROWS)])
    plsc.subcore_barrier()

    def body(h_hbm):
        pltpu.sync_copy(src3_hbm.at[pl.ds(s * BLOCKS_PER_TILE, BLOCKS_PER_TILE)],
                        src_v)
        pltpu.sync_copy(dst3_hbm.at[pl.ds(s * BLOCKS_PER_TILE, BLOCKS_PER_TILE)],
                        dst_v)

        def blk(j, carry):
            pltpu.async_copy(h_hbm.at[src_v.at[j]], rows0, sem_g).wait()
            pltpu.sync_copy(rows0, acc_sh.at[dst_v.at[j]], add=True)
            return carry

        lax.fori_loop(0, BLOCKS_PER_TILE, blk, 0)

    @pl.when(c == 0)
    def _():
        body(h0_hbm)

    @pl.when(c == 1)
    def _():
        body(h1_hbm)

    plsc.subcore_barrier()

    @pl.when(c == 0)
    def _():
        pltpu.sync_copy(acc_sh.at[pl.ds(s * ZROWS, ZROWS)],
                        out0_hbm.at[pl.ds(s * ZROWS, ZROWS)])

    @pl.when(c == 1)
    def _():
        pltpu.sync_copy(acc_sh.at[pl.ds(s * ZROWS, ZROWS)],
                        out1_hbm.at[pl.ds(s * ZROWS, ZROWS)])

  return _agg_sc


def _agg_call(h0, h1, src2, dst2, zeros):
    return _make_agg_sc()(h0, h1, src2, dst2, zeros)


def _mlp_body(h0, h1, a0, a1, w1, b1, w2, b2, out, ssum, ssq):
    x = jnp.concatenate([h0[...] + a0[...], h1[...] + a1[...]], axis=1)
    t = jnp.dot(x, w1[...], preferred_element_type=jnp.float32) + b1[...]
    t = jnp.maximum(t, 0.0)
    m = jnp.dot(t, w2[...], preferred_element_type=jnp.float32) + b2[...]
    m = jnp.maximum(m, 0.0)
    out[...] = m

    @pl.when(pl.program_id(0) == 0)
    def _():
        ssum[...] = jnp.zeros_like(ssum)
        ssq[...] = jnp.zeros_like(ssq)

    ssum[...] += jnp.sum(m, axis=0, keepdims=True)
    ssq[...] += jnp.sum(m * m, axis=0, keepdims=True)


def _mlp_call(h0, h1, a0, a1, w1, b1, w2, b2):
    row = pl.BlockSpec((R, DH), lambda i: (i, 0))
    full = pl.BlockSpec((D, D), lambda i: (0, 0))
    vec = pl.BlockSpec((1, D), lambda i: (0, 0))
    return pl.pallas_call(
        _mlp_body,
        grid=(N // R,),
        in_specs=[row, row, row, row, full, vec, full, vec],
        out_specs=[pl.BlockSpec((R, D), lambda i: (i, 0)), vec, vec],
        out_shape=[
            jax.ShapeDtypeStruct((N, D), jnp.float32),
            jax.ShapeDtypeStruct((1, D), jnp.float32),
            jax.ShapeDtypeStruct((1, D), jnp.float32),
        ],
    )(h0, h1, a0, a1, w1, b1, w2, b2)


def _bn_body(mraw, ssum, ssq, gamma, beta, batch, m0, m1, pool):
    mean = ssum[...] * (1.0 / N)
    var = ssq[...] * (1.0 / N) - mean * mean
    scale = gamma[...] * lax.rsqrt(var + BN_EPS)
    shift = beta[...] - mean * scale
    m = mraw[...] * scale + shift
    m0[...] = m[:, :DH]
    m1[...] = m[:, DH:]
    oh = (batch[...] == lax.broadcasted_iota(jnp.int32, (R, G), 1))
    oh = oh.astype(jnp.float32)

    @pl.when(pl.program_id(0) == 0)
    def _():
        pool[...] = jnp.zeros_like(pool)

    pool[...] += lax.dot_general(oh, m, (((0,), (0,)), ((), ())),
                                 preferred_element_type=jnp.float32)


def _bn_call(mraw, ssum, ssq, gamma, beta, batch2):
    vec = pl.BlockSpec((1, D), lambda i: (0, 0))
    return pl.pallas_call(
        _bn_body,
        grid=(N // R,),
        in_specs=[
            pl.BlockSpec((R, D), lambda i: (i, 0)),
            vec, vec, vec, vec,
            pl.BlockSpec((R, 1), lambda i: (i, 0)),
        ],
        out_specs=[
            pl.BlockSpec((R, DH), lambda i: (i, 0)),
            pl.BlockSpec((R, DH), lambda i: (i, 0)),
            pl.BlockSpec((G, D), lambda i: (0, 0)),
        ],
        out_shape=[
            jax.ShapeDtypeStruct((N, DH), jnp.float32),
            jax.ShapeDtypeStruct((N, DH), jnp.float32),
            jax.ShapeDtypeStruct((G, D), jnp.float32),
        ],
    )(mraw, ssum, ssq, gamma, beta, batch2)


def kernel(x, edge_index, batch,
           W1_0, b1_0, W2_0, b2_0, gamma_0, beta_0,
           W1_1, b1_1, W2_1, b2_1, gamma_1, beta_1,
           W1_2, b1_2, W2_2, b2_2, gamma_2, beta_2):
    src = edge_index[0].astype(jnp.int32)
    dst = edge_index[1].astype(jnp.int32)
    pad = EPAD - E
    src2 = jnp.concatenate([src, jnp.zeros((pad,), jnp.int32)]).reshape(-1, BLK)
    dst2 = jnp.concatenate([dst, jnp.full((pad,), N, jnp.int32)]).reshape(-1, BLK)
    zeros = jnp.zeros((NPAD, DH), jnp.float32)
    batch2 = batch.reshape(N, 1).astype(jnp.int32)

    params = [
        (W1_0, b1_0.reshape(1, D), W2_0, b2_0.reshape(1, D),
         gamma_0.reshape(1, D), beta_0.reshape(1, D)),
        (W1_1, b1_1.reshape(1, D), W2_1, b2_1.reshape(1, D),
         gamma_1.reshape(1, D), beta_1.reshape(1, D)),
        (W1_2, b1_2.reshape(1, D), W2_2, b2_2.reshape(1, D),
         gamma_2.reshape(1, D), beta_2.reshape(1, D)),
    ]

    h0 = x[:, :DH]
    h1 = x[:, DH:]
    halves = []
    pools = []
    for i in range(L):
        w1, b1, w2, b2, gmm, bta = params[i]
        a0, a1 = _agg_call(h0, h1, src2, dst2, zeros)
        mraw, ssum, ssq = _mlp_call(h0, h1, a0, a1, w1, b1, w2, b2)
        m0, m1, pool = _bn_call(mraw, ssum, ssq, gmm, bta, batch2)
        h0, h1 = m0, m1
        halves.extend([m0, m1])
        pools.append(pool)

    return jnp.concatenate(pools, axis=1), jnp.concatenate(halves, axis=1)


# speedup vs baseline: 3.0066x; 1.0009x over previous
import functools

import jax
import jax.numpy as jnp
from jax import lax
from jax.experimental import pallas as pl
from jax.experimental.pallas import tpu as pltpu
from jax.experimental.pallas import tpu_sc as plsc

N = 10000
E = 160000
D = 256
DH = 128
L = 3
G = 64
BN_EPS = 1e-5

NSUB = 16
BLK = 128
BLOCKS_PER_TILE = 80
EPT = BLOCKS_PER_TILE * BLK
EPAD = NSUB * EPT
NPAD = 10112
ZROWS = NPAD // NSUB

R = 2000


@functools.cache
def _make_agg_sc():
  mesh = plsc.VectorSubcoreMesh(core_axis_name="c", subcore_axis_name="s")

  @functools.partial(
      pl.kernel,
      mesh=mesh,
      out_type=[
          jax.ShapeDtypeStruct((NPAD, DH), jnp.float32),
          jax.ShapeDtypeStruct((NPAD, DH), jnp.float32),
      ],
      scratch_types=[
          pltpu.VMEM((BLOCKS_PER_TILE, BLK), jnp.int32),
          pltpu.VMEM((BLOCKS_PER_TILE, BLK), jnp.int32),
          pltpu.VMEM((BLK, DH), jnp.float32),
          pltpu.VMEM_SHARED((NPAD, DH), jnp.float32),
          pltpu.SemaphoreType.DMA,
      ],
  )
  def _agg_sc(h0_hbm, h1_hbm, src3_hbm, dst3_hbm, zeros_hbm, out0_hbm, out1_hbm,
              src_v, dst_v, rows0, acc_sh, sem_g):
    c = lax.axis_index("c")
    s = lax.axis_index("s")

    pltpu.sync_copy(zeros_hbm.at[pl.ds(s * ZROWS, ZROWS)],
                    acc_sh.at[pl.ds(s * ZROWS, ZROWS)])
    plsc.subcore_barrier()

    def body(h_hbm):
        pltpu.sync_copy(src3_hbm.at[pl.ds(s * BLOCKS_PER_TILE, BLOCKS_PER_TILE)],
                        src_v)
        pltpu.sync_copy(dst3_hbm.at[pl.ds(s * BLOCKS_PER_TILE, BLOCKS_PER_TILE)],
                        dst_v)

        def blk(j, carry):
            pltpu.async_copy(h_hbm.at[src_v.at[j]], rows0, sem_g).wait()
            pltpu.sync_copy(rows0, acc_sh.at[dst_v.at[j]], add=True)
            return carry

        lax.fori_loop(0, BLOCKS_PER_TILE, blk, 0)

    @pl.when(c == 0)
    def _():
        body(h0_hbm)

    @pl.when(c == 1)
    def _():
        body(h1_hbm)

    plsc.subcore_barrier()

    @pl.when(c == 0)
    def _():
        pltpu.sync_copy(acc_sh.at[pl.ds(s * ZROWS, ZROWS)],
                        out0_hbm.at[pl.ds(s * ZROWS, ZROWS)])

    @pl.when(c == 1)
    def _():
        pltpu.sync_copy(acc_sh.at[pl.ds(s * ZROWS, ZROWS)],
                        out1_hbm.at[pl.ds(s * ZROWS, ZROWS)])

  return _agg_sc


def _agg_call(h0, h1, src2, dst2, zeros):
    return _make_agg_sc()(h0, h1, src2, dst2, zeros)


def _mlp_body(h0, h1, a0, a1, w1, b1, w2, b2, out, ssum, ssq):
    x = jnp.concatenate([h0[...] + a0[...], h1[...] + a1[...]], axis=1)
    t = jnp.dot(x, w1[...], preferred_element_type=jnp.float32) + b1[...]
    t = jnp.maximum(t, 0.0)
    m = jnp.dot(t, w2[...], preferred_element_type=jnp.float32) + b2[...]
    m = jnp.maximum(m, 0.0)
    out[...] = m

    @pl.when(pl.program_id(0) == 0)
    def _():
        ssum[...] = jnp.zeros_like(ssum)
        ssq[...] = jnp.zeros_like(ssq)

    ssum[...] += jnp.sum(m, axis=0, keepdims=True)
    ssq[...] += jnp.sum(m * m, axis=0, keepdims=True)


def _mlp_call(h0, h1, a0, a1, w1, b1, w2, b2):
    row = pl.BlockSpec((R, DH), lambda i: (i, 0))
    full = pl.BlockSpec((D, D), lambda i: (0, 0))
    vec = pl.BlockSpec((1, D), lambda i: (0, 0))
    return pl.pallas_call(
        _mlp_body,
        grid=(N // R,),
        in_specs=[row, row, row, row, full, vec, full, vec],
        out_specs=[pl.BlockSpec((R, D), lambda i: (i, 0)), vec, vec],
        out_shape=[
            jax.ShapeDtypeStruct((N, D), jnp.float32),
            jax.ShapeDtypeStruct((1, D), jnp.float32),
            jax.ShapeDtypeStruct((1, D), jnp.float32),
        ],
    )(h0, h1, a0, a1, w1, b1, w2, b2)


def _bn_body(mraw, ssum, ssq, gamma, beta, batch, m0, m1, pool):
    mean = ssum[...] * (1.0 / N)
    var = ssq[...] * (1.0 / N) - mean * mean
    scale = gamma[...] * lax.rsqrt(var + BN_EPS)
    shift = beta[...] - mean * scale
    m = mraw[...] * scale + shift
    m0[...] = m[:, :DH]
    m1[...] = m[:, DH:]
    oh = (batch[...] == lax.broadcasted_iota(jnp.int32, (R, G), 1))
    oh = oh.astype(jnp.float32)

    @pl.when(pl.program_id(0) == 0)
    def _():
        pool[...] = jnp.zeros_like(pool)

    pool[...] += lax.dot_general(oh, m, (((0,), (0,)), ((), ())),
                                 preferred_element_type=jnp.float32)


def _bn_call(mraw, ssum, ssq, gamma, beta, batch2):
    vec = pl.BlockSpec((1, D), lambda i: (0, 0))
    return pl.pallas_call(
        _bn_body,
        grid=(N // R,),
        in_specs=[
            pl.BlockSpec((R, D), lambda i: (i, 0)),
            vec, vec, vec, vec,
            pl.BlockSpec((R, 1), lambda i: (i, 0)),
        ],
        out_specs=[
            pl.BlockSpec((R, DH), lambda i: (i, 0)),
            pl.BlockSpec((R, DH), lambda i: (i, 0)),
            pl.BlockSpec((G, D), lambda i: (0, 0)),
        ],
        out_shape=[
            jax.ShapeDtypeStruct((N, DH), jnp.float32),
            jax.ShapeDtypeStruct((N, DH), jnp.float32),
            jax.ShapeDtypeStruct((G, D), jnp.float32),
        ],
    )(mraw, ssum, ssq, gamma, beta, batch2)


def kernel(x, edge_index, batch,
           W1_0, b1_0, W2_0, b2_0, gamma_0, beta_0,
           W1_1, b1_1, W2_1, b2_1, gamma_1, beta_1,
           W1_2, b1_2, W2_2, b2_2, gamma_2, beta_2):
    src = edge_index[0].astype(jnp.int32)
    dst = edge_index[1].astype(jnp.int32)
    pad = EPAD - E
    pad_dst = N + jnp.arange(pad, dtype=jnp.int32) % (NPAD - N)
    src2 = jnp.concatenate([src, jnp.zeros((pad,), jnp.int32)]).reshape(-1, BLK)
    dst2 = jnp.concatenate([dst, pad_dst]).reshape(-1, BLK)
    zeros = jnp.zeros((NPAD, DH), jnp.float32)
    batch2 = batch.reshape(N, 1).astype(jnp.int32)

    params = [
        (W1_0, b1_0.reshape(1, D), W2_0, b2_0.reshape(1, D),
         gamma_0.reshape(1, D), beta_0.reshape(1, D)),
        (W1_1, b1_1.reshape(1, D), W2_1, b2_1.reshape(1, D),
         gamma_1.reshape(1, D), beta_1.reshape(1, D)),
        (W1_2, b1_2.reshape(1, D), W2_2, b2_2.reshape(1, D),
         gamma_2.reshape(1, D), beta_2.reshape(1, D)),
    ]

    h0 = x[:, :DH]
    h1 = x[:, DH:]
    halves = []
    pools = []
    for i in range(L):
        w1, b1, w2, b2, gmm, bta = params[i]
        a0, a1 = _agg_call(h0, h1, src2, dst2, zeros)
        mraw, ssum, ssq = _mlp_call(h0, h1, a0, a1, w1, b1, w2, b2)
        m0, m1, pool = _bn_call(mraw, ssum, ssq, gmm, bta, batch2)
        h0, h1 = m0, m1
        halves.extend([m0, m1])
        pools.append(pool)

    return jnp.concatenate(pools, axis=1), jnp.concatenate(halves, axis=1)


# speedup vs baseline: 3.1451x; 1.0460x over previous
import functools

import jax
import jax.numpy as jnp
from jax import lax
from jax.experimental import pallas as pl
from jax.experimental.pallas import tpu as pltpu
from jax.experimental.pallas import tpu_sc as plsc

N = 10000
E = 160000
D = 256
DH = 128
L = 3
G = 64
BN_EPS = 1e-5

NSUB = 16
BLK = 128
BLOCKS_PER_TILE = 80
EPT = BLOCKS_PER_TILE * BLK
EPAD = NSUB * EPT
NPAD = 10112
ZROWS = NPAD // NSUB

R = 2000


@functools.cache
def _make_agg_sc():
  mesh = plsc.VectorSubcoreMesh(core_axis_name="c", subcore_axis_name="s")

  @functools.partial(
      pl.kernel,
      mesh=mesh,
      out_type=[
          jax.ShapeDtypeStruct((NPAD, DH), jnp.float32),
          jax.ShapeDtypeStruct((NPAD, DH), jnp.float32),
      ],
      scratch_types=[
          pltpu.VMEM((BLOCKS_PER_TILE // 2, BLK), jnp.int32),
          pltpu.VMEM((BLOCKS_PER_TILE // 2, BLK), jnp.int32),
          pltpu.VMEM((BLK, DH), jnp.float32),
          pltpu.VMEM((BLK, DH), jnp.float32),
          pltpu.VMEM_SHARED((NPAD, DH), jnp.float32),
          pltpu.SemaphoreType.DMA,
          pltpu.SemaphoreType.DMA,
          pltpu.SemaphoreType.DMA,
          pltpu.SemaphoreType.DMA,
      ],
  )
  def _agg_sc(h0_hbm, h1_hbm, src3_hbm, dst3_hbm, zeros_hbm, out0_hbm, out1_hbm,
              src_v, dst_v, rows0, rows1, acc_sh, sem_g, sem_g2, sem_s, sem_s2):
    c = lax.axis_index("c")
    s = lax.axis_index("s")

    pltpu.sync_copy(zeros_hbm.at[pl.ds(s * ZROWS, ZROWS)],
                    acc_sh.at[pl.ds(s * ZROWS, ZROWS)])
    plsc.subcore_barrier()

    def body(h_hbm):
        half = BLOCKS_PER_TILE // 2
        for phase in range(2):
            off = s * BLOCKS_PER_TILE + phase * half
            pltpu.sync_copy(src3_hbm.at[pl.ds(off, half)], src_v)
            pltpu.sync_copy(dst3_hbm.at[pl.ds(off, half)], dst_v)

            def pair(i, carry):
                ga = pltpu.async_copy(h_hbm.at[src_v.at[2 * i]], rows0, sem_g)
                gb = pltpu.async_copy(h_hbm.at[src_v.at[2 * i + 1]], rows1,
                                      sem_g2)
                ga.wait()
                sa = pltpu.async_copy(rows0, acc_sh.at[dst_v.at[2 * i]],
                                      sem_s, add=True)
                gb.wait()
                sb = pltpu.async_copy(rows1, acc_sh.at[dst_v.at[2 * i + 1]],
                                      sem_s2, add=True)
                sa.wait()
                sb.wait()
                return carry

            lax.fori_loop(0, half // 2, pair, 0)

    @pl.when(c == 0)
    def _():
        body(h0_hbm)

    @pl.when(c == 1)
    def _():
        body(h1_hbm)

    plsc.subcore_barrier()

    @pl.when(c == 0)
    def _():
        pltpu.sync_copy(acc_sh.at[pl.ds(s * ZROWS, ZROWS)],
                        out0_hbm.at[pl.ds(s * ZROWS, ZROWS)])

    @pl.when(c == 1)
    def _():
        pltpu.sync_copy(acc_sh.at[pl.ds(s * ZROWS, ZROWS)],
                        out1_hbm.at[pl.ds(s * ZROWS, ZROWS)])

  return _agg_sc


def _agg_call(h0, h1, src2, dst2, zeros):
    return _make_agg_sc()(h0, h1, src2, dst2, zeros)


def _mlp_body(h0, h1, a0, a1, w1, b1, w2, b2, out, ssum, ssq):
    x = jnp.concatenate([h0[...] + a0[...], h1[...] + a1[...]], axis=1)
    t = jnp.dot(x, w1[...], preferred_element_type=jnp.float32) + b1[...]
    t = jnp.maximum(t, 0.0)
    m = jnp.dot(t, w2[...], preferred_element_type=jnp.float32) + b2[...]
    m = jnp.maximum(m, 0.0)
    out[...] = m

    @pl.when(pl.program_id(0) == 0)
    def _():
        ssum[...] = jnp.zeros_like(ssum)
        ssq[...] = jnp.zeros_like(ssq)

    ssum[...] += jnp.sum(m, axis=0, keepdims=True)
    ssq[...] += jnp.sum(m * m, axis=0, keepdims=True)


def _mlp_call(h0, h1, a0, a1, w1, b1, w2, b2):
    row = pl.BlockSpec((R, DH), lambda i: (i, 0))
    full = pl.BlockSpec((D, D), lambda i: (0, 0))
    vec = pl.BlockSpec((1, D), lambda i: (0, 0))
    return pl.pallas_call(
        _mlp_body,
        grid=(N // R,),
        in_specs=[row, row, row, row, full, vec, full, vec],
        out_specs=[pl.BlockSpec((R, D), lambda i: (i, 0)), vec, vec],
        out_shape=[
            jax.ShapeDtypeStruct((N, D), jnp.float32),
            jax.ShapeDtypeStruct((1, D), jnp.float32),
            jax.ShapeDtypeStruct((1, D), jnp.float32),
        ],
    )(h0, h1, a0, a1, w1, b1, w2, b2)


def _bn_body(mraw, ssum, ssq, gamma, beta, batch, m0, m1, pool):
    mean = ssum[...] * (1.0 / N)
    var = ssq[...] * (1.0 / N) - mean * mean
    scale = gamma[...] * lax.rsqrt(var + BN_EPS)
    shift = beta[...] - mean * scale
    m = mraw[...] * scale + shift
    m0[...] = m[:, :DH]
    m1[...] = m[:, DH:]
    oh = (batch[...] == lax.broadcasted_iota(jnp.int32, (R, G), 1))
    oh = oh.astype(jnp.float32)

    @pl.when(pl.program_id(0) == 0)
    def _():
        pool[...] = jnp.zeros_like(pool)

    pool[...] += lax.dot_general(oh, m, (((0,), (0,)), ((), ())),
                                 preferred_element_type=jnp.float32)


def _bn_call(mraw, ssum, ssq, gamma, beta, batch2):
    vec = pl.BlockSpec((1, D), lambda i: (0, 0))
    return pl.pallas_call(
        _bn_body,
        grid=(N // R,),
        in_specs=[
            pl.BlockSpec((R, D), lambda i: (i, 0)),
            vec, vec, vec, vec,
            pl.BlockSpec((R, 1), lambda i: (i, 0)),
        ],
        out_specs=[
            pl.BlockSpec((R, DH), lambda i: (i, 0)),
            pl.BlockSpec((R, DH), lambda i: (i, 0)),
            pl.BlockSpec((G, D), lambda i: (0, 0)),
        ],
        out_shape=[
            jax.ShapeDtypeStruct((N, DH), jnp.float32),
            jax.ShapeDtypeStruct((N, DH), jnp.float32),
            jax.ShapeDtypeStruct((G, D), jnp.float32),
        ],
    )(mraw, ssum, ssq, gamma, beta, batch2)


def kernel(x, edge_index, batch,
           W1_0, b1_0, W2_0, b2_0, gamma_0, beta_0,
           W1_1, b1_1, W2_1, b2_1, gamma_1, beta_1,
           W1_2, b1_2, W2_2, b2_2, gamma_2, beta_2):
    src = edge_index[0].astype(jnp.int32)
    dst = edge_index[1].astype(jnp.int32)
    pad = EPAD - E
    pad_dst = N + jnp.arange(pad, dtype=jnp.int32) % (NPAD - N)
    src2 = jnp.concatenate([src, jnp.zeros((pad,), jnp.int32)]).reshape(-1, BLK)
    dst2 = jnp.concatenate([dst, pad_dst]).reshape(-1, BLK)
    zeros = jnp.zeros((NPAD, DH), jnp.float32)
    batch2 = batch.reshape(N, 1).astype(jnp.int32)

    params = [
        (W1_0, b1_0.reshape(1, D), W2_0, b2_0.reshape(1, D),
         gamma_0.reshape(1, D), beta_0.reshape(1, D)),
        (W1_1, b1_1.reshape(1, D), W2_1, b2_1.reshape(1, D),
         gamma_1.reshape(1, D), beta_1.reshape(1, D)),
        (W1_2, b1_2.reshape(1, D), W2_2, b2_2.reshape(1, D),
         gamma_2.reshape(1, D), beta_2.reshape(1, D)),
    ]

    h0 = x[:, :DH]
    h1 = x[:, DH:]
    halves = []
    pools = []
    for i in range(L):
        w1, b1, w2, b2, gmm, bta = params[i]
        a0, a1 = _agg_call(h0, h1, src2, dst2, zeros)
        mraw, ssum, ssq = _mlp_call(h0, h1, a0, a1, w1, b1, w2, b2)
        m0, m1, pool = _bn_call(mraw, ssum, ssq, gmm, bta, batch2)
        h0, h1 = m0, m1
        halves.extend([m0, m1])
        pools.append(pool)

    return jnp.concatenate(pools, axis=1), jnp.concatenate(halves, axis=1)


# speedup vs baseline: 3.3496x; 1.0650x over previous
import functools

import jax
import jax.numpy as jnp
from jax import lax
from jax.experimental import pallas as pl
from jax.experimental.pallas import tpu as pltpu
from jax.experimental.pallas import tpu_sc as plsc

N = 10000
E = 160000
D = 256
DH = 128
L = 3
G = 64
BN_EPS = 1e-5

NSUB = 16
BLK = 128
BLOCKS_PER_TILE = 80
EPT = BLOCKS_PER_TILE * BLK
EPAD = NSUB * EPT
NPAD = 10112
ZROWS = NPAD // NSUB

R = 2000


@functools.cache
def _make_agg_sc():
  mesh = plsc.VectorSubcoreMesh(core_axis_name="c", subcore_axis_name="s")

  @functools.partial(
      pl.kernel,
      mesh=mesh,
      out_type=[
          jax.ShapeDtypeStruct((NPAD, DH), jnp.float32),
          jax.ShapeDtypeStruct((NPAD, DH), jnp.float32),
      ],
      scratch_types=[
          pltpu.VMEM((BLOCKS_PER_TILE // 2, BLK), jnp.int32),
          pltpu.VMEM((BLOCKS_PER_TILE // 2, BLK), jnp.int32),
          pltpu.VMEM((BLK, DH), jnp.float32),
          pltpu.VMEM((BLK, DH), jnp.float32),
          pltpu.VMEM_SHARED((NPAD, DH), jnp.float32),
          pltpu.SemaphoreType.DMA,
          pltpu.SemaphoreType.DMA,
          pltpu.SemaphoreType.DMA,
          pltpu.SemaphoreType.DMA,
      ],
  )
  def _agg_sc(h0_hbm, h1_hbm, src3_hbm, dst3_hbm, zeros_hbm, out0_hbm, out1_hbm,
              src_v, dst_v, rows0, rows1, acc_sh, sem_g, sem_g2, sem_s, sem_s2):
    c = lax.axis_index("c")
    s = lax.axis_index("s")

    pltpu.sync_copy(zeros_hbm.at[pl.ds(s * ZROWS, ZROWS)],
                    acc_sh.at[pl.ds(s * ZROWS, ZROWS)])
    plsc.subcore_barrier()

    def body(h_hbm):
        half = BLOCKS_PER_TILE // 2
        for phase in range(2):
            off = s * BLOCKS_PER_TILE + phase * half
            pltpu.sync_copy(src3_hbm.at[pl.ds(off, half)], src_v)
            pltpu.sync_copy(dst3_hbm.at[pl.ds(off, half)], dst_v)

            bufs = (rows0, rows1)
            gsems = (sem_g, sem_g2)
            ssems = (sem_s, sem_s2)
            pltpu.async_copy(h_hbm.at[src_v.at[0]], rows0, sem_g)

            def pair(i, carry):
                for b in range(2):
                    j = 2 * i + b
                    buf, ob = bufs[b], bufs[1 - b]
                    pltpu.make_async_copy(
                        h_hbm.at[src_v.at[j]], buf, gsems[b]).wait()
                    pltpu.async_copy(buf, acc_sh.at[dst_v.at[j]],
                                     ssems[b], add=True)

                    @pl.when(j >= 1)
                    def _():
                        pltpu.make_async_copy(
                            ob, acc_sh.at[dst_v.at[j - 1]],
                            ssems[1 - b]).wait()

                    @pl.when(j + 1 < half)
                    def _():
                        pltpu.async_copy(
                            h_hbm.at[src_v.at[j + 1]], ob, gsems[1 - b])
                return carry

            lax.fori_loop(0, half // 2, pair, 0)
            pltpu.make_async_copy(
                rows1, acc_sh.at[dst_v.at[half - 1]], sem_s2).wait()

    @pl.when(c == 0)
    def _():
        body(h0_hbm)

    @pl.when(c == 1)
    def _():
        body(h1_hbm)

    plsc.subcore_barrier()

    @pl.when(c == 0)
    def _():
        pltpu.sync_copy(acc_sh.at[pl.ds(s * ZROWS, ZROWS)],
                        out0_hbm.at[pl.ds(s * ZROWS, ZROWS)])

    @pl.when(c == 1)
    def _():
        pltpu.sync_copy(acc_sh.at[pl.ds(s * ZROWS, ZROWS)],
                        out1_hbm.at[pl.ds(s * ZROWS, ZROWS)])

  return _agg_sc


def _agg_call(h0, h1, src2, dst2, zeros):
    return _make_agg_sc()(h0, h1, src2, dst2, zeros)


def _mlp_body(h0, h1, a0, a1, w1, b1, w2, b2, out, ssum, ssq):
    x = jnp.concatenate([h0[...] + a0[...], h1[...] + a1[...]], axis=1)
    t = jnp.dot(x, w1[...], preferred_element_type=jnp.float32) + b1[...]
    t = jnp.maximum(t, 0.0)
    m = jnp.dot(t, w2[...], preferred_element_type=jnp.float32) + b2[...]
    m = jnp.maximum(m, 0.0)
    out[...] = m

    @pl.when(pl.program_id(0) == 0)
    def _():
        ssum[...] = jnp.zeros_like(ssum)
        ssq[...] = jnp.zeros_like(ssq)

    ssum[...] += jnp.sum(m, axis=0, keepdims=True)
    ssq[...] += jnp.sum(m * m, axis=0, keepdims=True)


def _mlp_call(h0, h1, a0, a1, w1, b1, w2, b2):
    row = pl.BlockSpec((R, DH), lambda i: (i, 0))
    full = pl.BlockSpec((D, D), lambda i: (0, 0))
    vec = pl.BlockSpec((1, D), lambda i: (0, 0))
    return pl.pallas_call(
        _mlp_body,
        grid=(N // R,),
        in_specs=[row, row, row, row, full, vec, full, vec],
        out_specs=[pl.BlockSpec((R, D), lambda i: (i, 0)), vec, vec],
        out_shape=[
            jax.ShapeDtypeStruct((N, D), jnp.float32),
            jax.ShapeDtypeStruct((1, D), jnp.float32),
            jax.ShapeDtypeStruct((1, D), jnp.float32),
        ],
    )(h0, h1, a0, a1, w1, b1, w2, b2)


def _bn_body(mraw, ssum, ssq, gamma, beta, batch, m0, m1, pool):
    mean = ssum[...] * (1.0 / N)
    var = ssq[...] * (1.0 / N) - mean * mean
    scale = gamma[...] * lax.rsqrt(var + BN_EPS)
    shift = beta[...] - mean * scale
    m = mraw[...] * scale + shift
    m0[...] = m[:, :DH]
    m1[...] = m[:, DH:]
    oh = (batch[...] == lax.broadcasted_iota(jnp.int32, (R, G), 1))
    oh = oh.astype(jnp.float32)

    @pl.when(pl.program_id(0) == 0)
    def _():
        pool[...] = jnp.zeros_like(pool)

    pool[...] += lax.dot_general(oh, m, (((0,), (0,)), ((), ())),
                                 preferred_element_type=jnp.float32)


def _bn_call(mraw, ssum, ssq, gamma, beta, batch2):
    vec = pl.BlockSpec((1, D), lambda i: (0, 0))
    return pl.pallas_call(
        _bn_body,
        grid=(N // R,),
        in_specs=[
            pl.BlockSpec((R, D), lambda i: (i, 0)),
            vec, vec, vec, vec,
            pl.BlockSpec((R, 1), lambda i: (i, 0)),
        ],
        out_specs=[
            pl.BlockSpec((R, DH), lambda i: (i, 0)),
            pl.BlockSpec((R, DH), lambda i: (i, 0)),
            pl.BlockSpec((G, D), lambda i: (0, 0)),
        ],
        out_shape=[
            jax.ShapeDtypeStruct((N, DH), jnp.float32),
            jax.ShapeDtypeStruct((N, DH), jnp.float32),
            jax.ShapeDtypeStruct((G, D), jnp.float32),
        ],
    )(mraw, ssum, ssq, gamma, beta, batch2)


def kernel(x, edge_index, batch,
           W1_0, b1_0, W2_0, b2_0, gamma_0, beta_0,
           W1_1, b1_1, W2_1, b2_1, gamma_1, beta_1,
           W1_2, b1_2, W2_2, b2_2, gamma_2, beta_2):
    src = edge_index[0].astype(jnp.int32)
    dst = edge_index[1].astype(jnp.int32)
    pad = EPAD - E
    pad_dst = N + jnp.arange(pad, dtype=jnp.int32) % (NPAD - N)
    src2 = jnp.concatenate([src, jnp.zeros((pad,), jnp.int32)]).reshape(-1, BLK)
    dst2 = jnp.concatenate([dst, pad_dst]).reshape(-1, BLK)
    zeros = jnp.zeros((NPAD, DH), jnp.float32)
    batch2 = batch.reshape(N, 1).astype(jnp.int32)

    params = [
        (W1_0, b1_0.reshape(1, D), W2_0, b2_0.reshape(1, D),
         gamma_0.reshape(1, D), beta_0.reshape(1, D)),
        (W1_1, b1_1.reshape(1, D), W2_1, b2_1.reshape(1, D),
         gamma_1.reshape(1, D), beta_1.reshape(1, D)),
        (W1_2, b1_2.reshape(1, D), W2_2, b2_2.reshape(1, D),
         gamma_2.reshape(1, D), beta_2.reshape(1, D)),
    ]

    h0 = x[:, :DH]
    h1 = x[:, DH:]
    halves = []
    pools = []
    for i in range(L):
        w1, b1, w2, b2, gmm, bta = params[i]
        a0, a1 = _agg_call(h0, h1, src2, dst2, zeros)
        mraw, ssum, ssq = _mlp_call(h0, h1, a0, a1, w1, b1, w2, b2)
        m0, m1, pool = _bn_call(mraw, ssum, ssq, gmm, bta, batch2)
        h0, h1 = m0, m1
        halves.extend([m0, m1])
        pools.append(pool)

    return jnp.concatenate(pools, axis=1), jnp.concatenate(halves, axis=1)
